# baseline (speedup 1.0000x reference)
import functools

import jax
import jax.numpy as jnp
from jax import lax
from jax.experimental import pallas as pl
from jax.experimental.pallas import tpu as pltpu
from jax.experimental.pallas import tpu_sc as plsc

START = 2000
NSUB = 8000
D = 128
NTOT = 10000
NE = 128000

NC = 2
NS = 16
NW = NC * NS
L = 16

EPW = 4096
EROWS = EPW // 128
EPAD = NW * EPW - NE
DEGP = 8192
ZLEN = DEGP // NS

_mesh = plsc.VectorSubcoreMesh(core_axis_name="c", subcore_axis_name="s")
_sc_params = pltpu.CompilerParams(use_tc_tiling_on_sc=False,
                                  needs_layout_passes=False)


@functools.partial(
    pl.kernel,
    out_type=(jax.ShapeDtypeStruct((DEGP,), jnp.float32),
              jax.ShapeDtypeStruct((DEGP,), jnp.float32)),
    mesh=_mesh,
    scratch_types=[
        pltpu.VMEM((EROWS, 128), jnp.int32),
        pltpu.VMEM((128,), jnp.float32),
        pltpu.VMEM((ZLEN,), jnp.float32),
        pltpu.VMEM_SHARED((DEGP,), jnp.float32),
        pltpu.SemaphoreType.DMA,
        pltpu.SemaphoreType.DMA,
    ],
    compiler_params=_sc_params,
)
def _sc_degree_kernel(adjp, deg0_out, deg1_out, eidx, ones, zbuf, shacc,
                      sem1, sem2):
    c = lax.axis_index("c")
    s = lax.axis_index("s")
    w = c * NS + s

    edma = pltpu.async_copy(adjp.at[w], eidx, sem1)

    zero16 = jnp.zeros((L,), jnp.float32)
    one16 = jnp.full((L,), 1.0, jnp.float32)
    for k in range(ZLEN // L):
        zbuf[pl.ds(L * k, L)] = zero16
    for k in range(128 // L):
        ones[pl.ds(L * k, L)] = one16
    pltpu.sync_copy(zbuf, shacc.at[pl.ds(s * ZLEN, ZLEN)])

    edma.wait()
    plsc.subcore_barrier()

    descs = [pltpu.async_copy(ones, shacc.at[eidx.at[j]], sem2, add=True)
             for j in range(EROWS)]
    for d_ in descs:
        d_.wait()
    plsc.subcore_barrier()

    @pl.when((s == 0) & (c == 0))
    def _():
        pltpu.sync_copy(shacc, deg0_out)

    @pl.when((s == 0) & (c == 1))
    def _():
        pltpu.sync_copy(shacc, deg1_out)


_CS_BLK = 1000


def _tc_colsum_body(x_ref, o_ref):
    i = pl.program_id(0)

    @pl.when(i == 0)
    def _():
        o_ref[...] = jnp.zeros_like(o_ref)

    o_ref[...] += jnp.sum(x_ref[...], axis=0, keepdims=True)


_tc_colsum = pl.pallas_call(
    _tc_colsum_body,
    grid=(NSUB // _CS_BLK,),
    in_specs=[pl.BlockSpec((_CS_BLK, D), lambda i: (i + START // _CS_BLK, 0))],
    out_specs=pl.BlockSpec((1, D), lambda i: (0, 0)),
    out_shape=jax.ShapeDtypeStruct((1, D), jnp.float32),
)


_UP_BLK = 400
_HEAD_BLKS = START // _UP_BLK


def _tc_update_body(x_ref, s_ref, d0_ref, d1_ref, o_ref):
    i = pl.program_id(0)

    @pl.when(i < _HEAD_BLKS)
    def _():
        o_ref[...] = x_ref[...]

    @pl.when(i >= _HEAD_BLKS)
    def _():
        t = 1.0 / (1.0 + d0_ref[...] + d1_ref[...])
        s = s_ref[...]
        x = x_ref[...]
        o_ref[...] = (x + s) * (1.0 - s * t)


_tc_update = pl.pallas_call(
    _tc_update_body,
    grid=(NTOT // _UP_BLK,),
    in_specs=[
        pl.BlockSpec((_UP_BLK, D), lambda i: (i, 0)),
        pl.BlockSpec((1, D), lambda i: (0, 0)),
        pl.BlockSpec((_UP_BLK, 1), lambda i: (jnp.maximum(i - _HEAD_BLKS, 0), 0)),
        pl.BlockSpec((_UP_BLK, 1), lambda i: (jnp.maximum(i - _HEAD_BLKS, 0), 0)),
    ],
    out_specs=pl.BlockSpec((_UP_BLK, D), lambda i: (i, 0)),
    out_shape=jax.ShapeDtypeStruct((NTOT, D), jnp.float32),
)


def kernel(all_node_embedding, adj_nonzero_rows):
    adjp = jnp.concatenate(
        [adj_nonzero_rows.astype(jnp.int32),
         jnp.full((EPAD,), NSUB, jnp.int32)]).reshape(NW, EROWS, 128)
    d0, d1 = _sc_degree_kernel(adjp)
    s = _tc_colsum(all_node_embedding)
    return _tc_update(all_node_embedding, s,
                      d0.reshape(DEGP, 1), d1.reshape(DEGP, 1))


# speedup vs baseline: 1.1884x; 1.1884x over previous
import functools

import jax
import jax.numpy as jnp
from jax import lax
from jax.experimental import pallas as pl
from jax.experimental.pallas import tpu as pltpu
from jax.experimental.pallas import tpu_sc as plsc

START = 2000
NSUB = 8000
D = 128
NTOT = 10000
NE = 128000

NC = 2
NS = 16
NW = NC * NS
L = 16

EPW = NE // NW
EPWP = 4096
EROWS = EPWP // 128
DEGP = 8192
ZLEN = DEGP // NS

_mesh = plsc.VectorSubcoreMesh(core_axis_name="c", subcore_axis_name="s")
_sc_params = pltpu.CompilerParams(use_tc_tiling_on_sc=False,
                                  needs_layout_passes=False)


@functools.partial(
    pl.kernel,
    out_type=(jax.ShapeDtypeStruct((DEGP,), jnp.float32),
              jax.ShapeDtypeStruct((DEGP,), jnp.float32)),
    mesh=_mesh,
    scratch_types=[
        pltpu.VMEM((EPWP,), jnp.int32),
        pltpu.VMEM((128,), jnp.float32),
        pltpu.VMEM((ZLEN,), jnp.float32),
        pltpu.VMEM_SHARED((DEGP,), jnp.float32),
        pltpu.SemaphoreType.DMA,
        pltpu.SemaphoreType.DMA,
    ],
    compiler_params=_sc_params,
)
def _sc_degree_kernel(adj, deg0_out, deg1_out, eidx, ones, zbuf, shacc,
                      sem1, sem2):
    c = lax.axis_index("c")
    s = lax.axis_index("s")
    w = c * NS + s

    edma = pltpu.async_copy(adj.at[pl.ds(w * EPW, EPW)],
                            eidx.at[pl.ds(0, EPW)], sem1)

    zero16 = jnp.zeros((L,), jnp.float32)
    one16 = jnp.full((L,), 1.0, jnp.float32)
    pad16 = jnp.full((L,), NSUB, jnp.int32)
    for k in range(ZLEN // L):
        zbuf[pl.ds(L * k, L)] = zero16
    for k in range(128 // L):
        ones[pl.ds(L * k, L)] = one16
    for k in range((EPWP - EPW) // L):
        eidx[pl.ds(EPW + L * k, L)] = pad16
    pltpu.sync_copy(zbuf, shacc.at[pl.ds(s * ZLEN, ZLEN)])

    edma.wait()
    plsc.subcore_barrier()

    descs = [pltpu.async_copy(ones, shacc.at[eidx.at[pl.ds(128 * j, 128)]],
                              sem2, add=True)
             for j in range(EROWS)]
    for d_ in descs:
        d_.wait()
    plsc.subcore_barrier()

    @pl.when((s == 0) & (c == 0))
    def _():
        pltpu.sync_copy(shacc, deg0_out)

    @pl.when((s == 0) & (c == 1))
    def _():
        pltpu.sync_copy(shacc, deg1_out)


_CS_BLK = 1000


def _tc_colsum_body(x_ref, o_ref):
    i = pl.program_id(0)

    @pl.when(i == 0)
    def _():
        o_ref[...] = jnp.zeros_like(o_ref)

    o_ref[...] += jnp.sum(x_ref[...], axis=0, keepdims=True)


_tc_colsum = pl.pallas_call(
    _tc_colsum_body,
    grid=(NSUB // _CS_BLK,),
    in_specs=[pl.BlockSpec((_CS_BLK, D), lambda i: (i + START // _CS_BLK, 0))],
    out_specs=pl.BlockSpec((1, D), lambda i: (0, 0)),
    out_shape=jax.ShapeDtypeStruct((1, D), jnp.float32),
)


_UP_BLK = 400
_HEAD_BLKS = START // _UP_BLK


def _tc_update_body(x_ref, s_ref, d0_ref, d1_ref, o_ref):
    i = pl.program_id(0)

    @pl.when(i < _HEAD_BLKS)
    def _():
        o_ref[...] = x_ref[...]

    @pl.when(i >= _HEAD_BLKS)
    def _():
        t64 = 1.0 / (1.0 + d0_ref[...] + d1_ref[...])
        p = lax.broadcasted_iota(jnp.int32, (_UP_BLK, 1), 0) + \
            (i - _HEAD_BLKS) * _UP_BLK
        rowsel = (p // D == lax.broadcasted_iota(jnp.int32, (1, DEGP // D), 1))
        b = jnp.dot(rowsel.astype(jnp.float32), t64,
                    preferred_element_type=jnp.float32)
        lanesel = (p % D == lax.broadcasted_iota(jnp.int32, (1, D), 1))
        t = jnp.sum(jnp.where(lanesel, b, 0.0), axis=1, keepdims=True)
        s = s_ref[...]
        x = x_ref[...]
        o_ref[...] = (x + s) * (1.0 - s * t)


_tc_update = pl.pallas_call(
    _tc_update_body,
    grid=(NTOT // _UP_BLK,),
    in_specs=[
        pl.BlockSpec((_UP_BLK, D), lambda i: (i, 0)),
        pl.BlockSpec((1, D), lambda i: (0, 0)),
        pl.BlockSpec((DEGP // D, D), lambda i: (0, 0)),
        pl.BlockSpec((DEGP // D, D), lambda i: (0, 0)),
    ],
    out_specs=pl.BlockSpec((_UP_BLK, D), lambda i: (i, 0)),
    out_shape=jax.ShapeDtypeStruct((NTOT, D), jnp.float32),
)


def kernel(all_node_embedding, adj_nonzero_rows):
    d0, d1 = _sc_degree_kernel(adj_nonzero_rows.astype(jnp.int32))
    s = _tc_colsum(all_node_embedding)
    return _tc_update(all_node_embedding, s,
                      d0.reshape(DEGP // D, D), d1.reshape(DEGP // D, D))
